# baseline (speedup 1.0000x reference)
import functools

import jax
import jax.numpy as jnp
from jax.experimental import pallas as pl
from jax.experimental.pallas import tpu as pltpu

N = 50000
E = 400000
D = 128
H = 128
O = 128

ROW_BLOCK = 1024
N_PAD = 50176


def _fused_layer_kernel(sa_ref, sb_ref, inva_ref, invb_ref, x_ref,
                        wla_ref, wlb_ref, wr_ref, b_ref, out_ref):
    ma = sa_ref[...] * inva_ref[...]
    mb = sb_ref[...] * invb_ref[...]
    acc = jnp.dot(ma, wla_ref[...], preferred_element_type=jnp.float32)
    acc += jnp.dot(mb, wlb_ref[...], preferred_element_type=jnp.float32)
    acc += jnp.dot(x_ref[...], wr_ref[...], preferred_element_type=jnp.float32)
    acc += b_ref[...]
    out_ref[...] = jnp.maximum(acc, 0.0)


def _fused_final_kernel(sa_ref, sb_ref, inva_ref, invb_ref, x_ref,
                        wla_ref, wlb_ref, wr_ref, b_ref,
                        wlin_ref, blin_ref, out_ref):
    ma = sa_ref[...] * inva_ref[...]
    mb = sb_ref[...] * invb_ref[...]
    acc = jnp.dot(ma, wla_ref[...], preferred_element_type=jnp.float32)
    acc += jnp.dot(mb, wlb_ref[...], preferred_element_type=jnp.float32)
    acc += jnp.dot(x_ref[...], wr_ref[...], preferred_element_type=jnp.float32)
    acc += b_ref[...]
    h = jnp.maximum(acc, 0.0)
    out_ref[...] = jnp.dot(h, wlin_ref[...], preferred_element_type=jnp.float32) + blin_ref[...]


def _row_spec():
    return pl.BlockSpec((ROW_BLOCK, 128), lambda i: (i, 0))


def _inv_spec():
    return pl.BlockSpec((ROW_BLOCK, 1), lambda i: (i, 0))


def _w_spec():
    return pl.BlockSpec((128, 128), lambda i: (0, 0))


def _b_spec():
    return pl.BlockSpec((1, 128), lambda i: (0, 0))


def _fused_layer(sa, sb, inva, invb, x, wla, wlb, wr, b):
    grid = (N_PAD // ROW_BLOCK,)
    return pl.pallas_call(
        _fused_layer_kernel,
        grid=grid,
        in_specs=[_row_spec(), _row_spec(), _inv_spec(), _inv_spec(),
                  _row_spec(), _w_spec(), _w_spec(), _w_spec(), _b_spec()],
        out_specs=_row_spec(),
        out_shape=jax.ShapeDtypeStruct((N_PAD, H), jnp.float32),
    )(sa, sb, inva, invb, x, wla, wlb, wr, b)


def _fused_final(sa, sb, inva, invb, x, wla, wlb, wr, b, wlin, blin):
    grid = (N_PAD // ROW_BLOCK,)
    return pl.pallas_call(
        _fused_final_kernel,
        grid=grid,
        in_specs=[_row_spec(), _row_spec(), _inv_spec(), _inv_spec(),
                  _row_spec(), _w_spec(), _w_spec(), _w_spec(), _b_spec(),
                  _w_spec(), _b_spec()],
        out_specs=_row_spec(),
        out_shape=jax.ShapeDtypeStruct((N_PAD, O), jnp.float32),
    )(sa, sb, inva, invb, x, wla, wlb, wr, b, wlin, blin)


def kernel(x_note, edge_index_to, edge_index_rev_to, Wl0_to, bl0_to, Wr0_to,
           Wl0_rev, bl0_rev, Wr0_rev, Wl1_to, bl1_to, Wr1_to, Wl1_rev,
           bl1_rev, Wr1_rev, W_lin, b_lin):
    src_to, dst_to = edge_index_to[0], edge_index_to[1]
    src_rv, dst_rv = edge_index_rev_to[0], edge_index_rev_to[1]

    ones = jnp.ones((E,), jnp.float32)
    cnt_to = jax.ops.segment_sum(ones, dst_to, num_segments=N)
    cnt_rv = jax.ops.segment_sum(ones, dst_rv, num_segments=N)
    inv_to = (1.0 / jnp.clip(cnt_to, 1.0, None))[:, None]
    inv_rv = (1.0 / jnp.clip(cnt_rv, 1.0, None))[:, None]
    inv_to_p = jnp.pad(inv_to, ((0, N_PAD - N), (0, 0)))
    inv_rv_p = jnp.pad(inv_rv, ((0, N_PAD - N), (0, 0)))

    x_p = jnp.pad(x_note, ((0, N_PAD - N), (0, 0)))

    s0_to = jax.ops.segment_sum(jnp.take(x_note, src_to, axis=0), dst_to,
                                num_segments=N)
    s0_rv = jax.ops.segment_sum(jnp.take(x_note, src_rv, axis=0), dst_rv,
                                num_segments=N)
    s0_to = jnp.pad(s0_to, ((0, N_PAD - N), (0, 0)))
    s0_rv = jnp.pad(s0_rv, ((0, N_PAD - N), (0, 0)))

    b0 = (bl0_to + bl0_rev)[None, :]
    h = _fused_layer(s0_to, s0_rv, inv_to_p, inv_rv_p, x_p,
                     Wl0_to, Wl0_rev, Wr0_to + Wr0_rev, b0)
    h_n = h[:N]

    s1_to = jax.ops.segment_sum(jnp.take(h_n, src_to, axis=0), dst_to,
                                num_segments=N)
    s1_rv = jax.ops.segment_sum(jnp.take(h_n, src_rv, axis=0), dst_rv,
                                num_segments=N)
    s1_to = jnp.pad(s1_to, ((0, N_PAD - N), (0, 0)))
    s1_rv = jnp.pad(s1_rv, ((0, N_PAD - N), (0, 0)))

    b1 = (bl1_to + bl1_rev)[None, :]
    out = _fused_final(s1_to, s1_rv, inv_to_p, inv_rv_p, h,
                       Wl1_to, Wl1_rev, Wr1_to + Wr1_rev, b1,
                       W_lin, b_lin[None, :])
    return out[:N]


# speedup vs baseline: 1.3259x; 1.3259x over previous
import functools

import jax
import jax.numpy as jnp
from jax import lax
from jax.experimental import pallas as pl
from jax.experimental.pallas import tpu as pltpu
from jax.experimental.pallas import tpu_sc as plsc

N = 50000
E = 400000

NC = 2
NS = 16
LANES = 16

CHUNK = 12544
N_PAD = 4 * CHUNK
SHARE = CHUNK // NS
GARB = CHUNK
ACC_ROWS = CHUNK + 16

EB = 128
E_PER_TILE = 25088
E_PAD = NS * E_PER_TILE
NBLK = E_PER_TILE // EB

ZROWS = 112

ROW_BLOCK = 1024


def _sc_agg_body(x_ref, src_ref, dst_ref, agg_ref, cnt_ref,
                 src_v, dst_v, dloc_v, ones_v, cstage_v, rows_v,
                 acc_sh, cnt_sh, sem):
    c = lax.axis_index("c")
    t = lax.axis_index("s")

    def init16(i, _):
        off = pl.multiple_of(i * 16, 16)
        ones_v[pl.ds(off, 16)] = jnp.full((16,), 1.0, jnp.float32)
        return 0
    lax.fori_loop(0, EB // 16, init16, 0)

    for p in range(2):
        chunk = 2 * p + c
        lo = chunk * CHUNK

        def zrow(i, _):
            r = i // 8
            off = pl.multiple_of((i % 8) * 16, 16)
            rows_v[r, pl.ds(off, 16)] = jnp.zeros((16,), jnp.float32)
            return 0
        lax.fori_loop(0, EB * 8, zrow, 0)

        def zc(i, _):
            off = pl.multiple_of(i * 16, 16)
            cstage_v[pl.ds(off, 16)] = jnp.zeros((16,), jnp.float32)
            return 0
        lax.fori_loop(0, SHARE // 16, zc, 0)

        for z in range(SHARE // EB):
            pltpu.sync_copy(rows_v, acc_sh.at[pl.ds(t * SHARE + z * EB, EB)])
        pltpu.sync_copy(rows_v.at[pl.ds(0, SHARE % EB)],
                        acc_sh.at[pl.ds(t * SHARE + (SHARE // EB) * EB,
                                        SHARE % EB)])
        pltpu.sync_copy(cstage_v, cnt_sh.at[pl.ds(t * SHARE, SHARE)])
        plsc.subcore_barrier()

        def blk(b, _):
            base = pl.multiple_of(t * E_PER_TILE + b * EB, EB)
            pltpu.sync_copy(src_ref.at[pl.ds(base, EB)], src_v)
            pltpu.sync_copy(dst_ref.at[pl.ds(base, EB)], dst_v)

            def lane(j, _):
                off = pl.multiple_of(j * 16, 16)
                dv = dst_v[pl.ds(off, 16)]
                inside = (dv >= lo) & (dv < lo + CHUNK)
                dloc_v[pl.ds(off, 16)] = jnp.where(inside, dv - lo, GARB)
                return 0
            lax.fori_loop(0, EB // 16, lane, 0)

            pltpu.async_copy(x_ref.at[src_v], rows_v, sem).wait()
            pltpu.sync_copy(rows_v, acc_sh.at[dloc_v], add=True)
            pltpu.sync_copy(ones_v, cnt_sh.at[dloc_v], add=True)
            return 0
        lax.fori_loop(0, NBLK, blk, 0)
        plsc.subcore_barrier()

        row0 = chunk * CHUNK + t * SHARE
        pltpu.sync_copy(acc_sh.at[pl.ds(t * SHARE, SHARE)],
                        agg_ref.at[pl.ds(row0, SHARE)])
        pltpu.sync_copy(cnt_sh.at[pl.ds(t * SHARE, SHARE)], cstage_v)
        pltpu.sync_copy(cstage_v, cnt_ref.at[pl.ds(row0, SHARE)])


def _make_sc_agg(table_rows):
    mesh = plsc.VectorSubcoreMesh(core_axis_name="c", subcore_axis_name="s",
                                  num_cores=NC, num_subcores=NS)
    return pl.kernel(
        _sc_agg_body,
        out_type=(jax.ShapeDtypeStruct((N_PAD, 128), jnp.float32),
                  jax.ShapeDtypeStruct((N_PAD,), jnp.float32)),
        mesh=mesh,
        scratch_types=[
            pltpu.VMEM((EB,), jnp.int32),
            pltpu.VMEM((EB,), jnp.int32),
            pltpu.VMEM((EB,), jnp.int32),
            pltpu.VMEM((EB,), jnp.float32),
            pltpu.VMEM((SHARE,), jnp.float32),
            pltpu.VMEM((EB, 128), jnp.float32),
            pltpu.VMEM_SHARED((ACC_ROWS, 128), jnp.float32),
            pltpu.VMEM_SHARED((ACC_ROWS,), jnp.float32),
            pltpu.SemaphoreType.DMA,
        ],
    )


def _fused_layer_kernel(sa_ref, sb_ref, inva_ref, invb_ref, x_ref,
                        wla_ref, wlb_ref, wr_ref, b_ref, out_ref):
    ma = sa_ref[...] * inva_ref[...]
    mb = sb_ref[...] * invb_ref[...]
    acc = jnp.dot(ma, wla_ref[...], preferred_element_type=jnp.float32)
    acc += jnp.dot(mb, wlb_ref[...], preferred_element_type=jnp.float32)
    acc += jnp.dot(x_ref[...], wr_ref[...], preferred_element_type=jnp.float32)
    acc += b_ref[...]
    out_ref[...] = jnp.maximum(acc, 0.0)


def _fused_final_kernel(sa_ref, sb_ref, inva_ref, invb_ref, x_ref,
                        wla_ref, wlb_ref, wr_ref, b_ref,
                        wlin_ref, blin_ref, out_ref):
    ma = sa_ref[...] * inva_ref[...]
    mb = sb_ref[...] * invb_ref[...]
    acc = jnp.dot(ma, wla_ref[...], preferred_element_type=jnp.float32)
    acc += jnp.dot(mb, wlb_ref[...], preferred_element_type=jnp.float32)
    acc += jnp.dot(x_ref[...], wr_ref[...], preferred_element_type=jnp.float32)
    acc += b_ref[...]
    h = jnp.maximum(acc, 0.0)
    out_ref[...] = jnp.dot(h, wlin_ref[...],
                           preferred_element_type=jnp.float32) + blin_ref[...]


def _row_spec():
    return pl.BlockSpec((ROW_BLOCK, 128), lambda i: (i, 0))


def _inv_spec():
    return pl.BlockSpec((ROW_BLOCK, 1), lambda i: (i, 0))


def _w_spec():
    return pl.BlockSpec((128, 128), lambda i: (0, 0))


def _b_spec():
    return pl.BlockSpec((1, 128), lambda i: (0, 0))


def _fused_layer(sa, sb, inva, invb, x, wla, wlb, wr, b):
    return pl.pallas_call(
        _fused_layer_kernel,
        grid=(N_PAD // ROW_BLOCK,),
        in_specs=[_row_spec(), _row_spec(), _inv_spec(), _inv_spec(),
                  _row_spec(), _w_spec(), _w_spec(), _w_spec(), _b_spec()],
        out_specs=_row_spec(),
        out_shape=jax.ShapeDtypeStruct((N_PAD, 128), jnp.float32),
    )(sa, sb, inva, invb, x, wla, wlb, wr, b)


def _fused_final(sa, sb, inva, invb, x, wla, wlb, wr, b, wlin, blin):
    return pl.pallas_call(
        _fused_final_kernel,
        grid=(N_PAD // ROW_BLOCK,),
        in_specs=[_row_spec(), _row_spec(), _inv_spec(), _inv_spec(),
                  _row_spec(), _w_spec(), _w_spec(), _w_spec(), _b_spec(),
                  _w_spec(), _b_spec()],
        out_specs=_row_spec(),
        out_shape=jax.ShapeDtypeStruct((N_PAD, 128), jnp.float32),
    )(sa, sb, inva, invb, x, wla, wlb, wr, b, wlin, blin)


def kernel(x_note, edge_index_to, edge_index_rev_to, Wl0_to, bl0_to, Wr0_to,
           Wl0_rev, bl0_rev, Wr0_rev, Wl1_to, bl1_to, Wr1_to, Wl1_rev,
           bl1_rev, Wr1_rev, W_lin, b_lin):
    pad_e = E_PAD - E
    src_to = jnp.concatenate([edge_index_to[0], jnp.zeros((pad_e,), jnp.int32)])
    dst_to = jnp.concatenate([edge_index_to[1],
                              jnp.full((pad_e,), 1 << 30, jnp.int32)])
    src_rv = jnp.concatenate([edge_index_rev_to[0],
                              jnp.zeros((pad_e,), jnp.int32)])
    dst_rv = jnp.concatenate([edge_index_rev_to[1],
                              jnp.full((pad_e,), 1 << 30, jnp.int32)])

    agg0 = _make_sc_agg(N)
    agg1 = _make_sc_agg(N_PAD)

    s0_to, cnt_to = agg0(x_note, src_to, dst_to)
    s0_rv, cnt_rv = agg0(x_note, src_rv, dst_rv)

    inv_to = (1.0 / jnp.clip(cnt_to, 1.0, None))[:, None]
    inv_rv = (1.0 / jnp.clip(cnt_rv, 1.0, None))[:, None]

    x_p = jnp.pad(x_note, ((0, N_PAD - N), (0, 0)))
    b0 = (bl0_to + bl0_rev)[None, :]
    h = _fused_layer(s0_to, s0_rv, inv_to, inv_rv, x_p,
                     Wl0_to, Wl0_rev, Wr0_to + Wr0_rev, b0)

    s1_to, _ = agg1(h, src_to, dst_to)
    s1_rv, _ = agg1(h, src_rv, dst_rv)

    b1 = (bl1_to + bl1_rev)[None, :]
    out = _fused_final(s1_to, s1_rv, inv_to, inv_rv, h,
                       Wl1_to, Wl1_rev, Wr1_to + Wr1_rev, b1,
                       W_lin, b_lin[None, :])
    return out[:N]


# speedup vs baseline: 1.3428x; 1.0128x over previous
import functools

import jax
import jax.numpy as jnp
from jax import lax
from jax.experimental import pallas as pl
from jax.experimental.pallas import tpu as pltpu
from jax.experimental.pallas import tpu_sc as plsc

N = 50000
E = 400000

NC = 2
NS = 16
LANES = 16

CHUNK = 12544
N_PAD = 4 * CHUNK
SHARE = CHUNK // NS
GARB = CHUNK
ACC_ROWS = CHUNK + 16

EB = 96
E_PER_TILE = 25152
E_PAD = NS * E_PER_TILE
NBLK = E_PER_TILE // EB

ZROWS = 112

ROW_BLOCK = 1024


def _sc_agg_body(x_ref, src_ref, dst_ref, agg_ref, cnt_ref,
                 src_a, src_b, dst_a, dst_b, dloc_a, dloc_b,
                 ones_v, cstage_v, rows_a, rows_b,
                 acc_sh, cnt_sh, sem_a, sem_b):
    c = lax.axis_index("c")
    t = lax.axis_index("s")

    def init16(i, _):
        off = pl.multiple_of(i * 16, 16)
        ones_v[pl.ds(off, 16)] = jnp.full((16,), 1.0, jnp.float32)
        return 0
    lax.fori_loop(0, EB // 16, init16, 0)

    for p in range(2):
        chunk = 2 * p + c
        lo = chunk * CHUNK

        def zrow(i, _):
            r = i // 8
            off = pl.multiple_of((i % 8) * 16, 16)
            rows_a[r, pl.ds(off, 16)] = jnp.zeros((16,), jnp.float32)
            return 0
        lax.fori_loop(0, EB * 8, zrow, 0)

        def zc(i, _):
            off = pl.multiple_of(i * 16, 16)
            cstage_v[pl.ds(off, 16)] = jnp.zeros((16,), jnp.float32)
            return 0
        lax.fori_loop(0, SHARE // 16, zc, 0)

        for z in range(SHARE // EB):
            pltpu.sync_copy(rows_a, acc_sh.at[pl.ds(t * SHARE + z * EB, EB)])
        pltpu.sync_copy(rows_a.at[pl.ds(0, SHARE % EB)],
                        acc_sh.at[pl.ds(t * SHARE + (SHARE // EB) * EB,
                                        SHARE % EB)])
        pltpu.sync_copy(cstage_v, cnt_sh.at[pl.ds(t * SHARE, SHARE)])
        plsc.subcore_barrier()

        def stage(b, sv, dv, dl):
            base = pl.multiple_of(t * E_PER_TILE + b * EB, EB)
            pltpu.sync_copy(src_ref.at[pl.ds(base, EB)], sv)
            pltpu.sync_copy(dst_ref.at[pl.ds(base, EB)], dv)

            def lane(j, _):
                off = pl.multiple_of(j * 16, 16)
                dvv = dv[pl.ds(off, 16)]
                inside = (dvv >= lo) & (dvv < lo + CHUNK)
                dl[pl.ds(off, 16)] = jnp.where(inside, dvv - lo, GARB)
                return 0
            lax.fori_loop(0, EB // 16, lane, 0)

        def scatter(rv, dl):
            pltpu.sync_copy(rv, acc_sh.at[dl], add=True)
            pltpu.sync_copy(ones_v, cnt_sh.at[dl], add=True)

        stage(0, src_a, dst_a, dloc_a)
        pltpu.async_copy(x_ref.at[src_a], rows_a, sem_a)

        def pair(g, _):
            stage(2 * g + 1, src_b, dst_b, dloc_b)
            pltpu.async_copy(x_ref.at[src_b], rows_b, sem_b)
            pltpu.make_async_copy(x_ref.at[src_a], rows_a, sem_a).wait()
            scatter(rows_a, dloc_a)

            @pl.when(g < NBLK // 2 - 1)
            def _():
                stage(2 * g + 2, src_a, dst_a, dloc_a)
                pltpu.async_copy(x_ref.at[src_a], rows_a, sem_a)

            pltpu.make_async_copy(x_ref.at[src_b], rows_b, sem_b).wait()
            scatter(rows_b, dloc_b)
            return 0
        lax.fori_loop(0, NBLK // 2, pair, 0)
        plsc.subcore_barrier()

        row0 = chunk * CHUNK + t * SHARE
        pltpu.sync_copy(acc_sh.at[pl.ds(t * SHARE, SHARE)],
                        agg_ref.at[pl.ds(row0, SHARE)])
        pltpu.sync_copy(cnt_sh.at[pl.ds(t * SHARE, SHARE)], cstage_v)
        pltpu.sync_copy(cstage_v, cnt_ref.at[pl.ds(row0, SHARE)])


def _make_sc_agg(table_rows):
    mesh = plsc.VectorSubcoreMesh(core_axis_name="c", subcore_axis_name="s",
                                  num_cores=NC, num_subcores=NS)
    return pl.kernel(
        _sc_agg_body,
        out_type=(jax.ShapeDtypeStruct((N_PAD, 128), jnp.float32),
                  jax.ShapeDtypeStruct((N_PAD,), jnp.float32)),
        mesh=mesh,
        scratch_types=[
            pltpu.VMEM((EB,), jnp.int32),
            pltpu.VMEM((EB,), jnp.int32),
            pltpu.VMEM((EB,), jnp.int32),
            pltpu.VMEM((EB,), jnp.int32),
            pltpu.VMEM((EB,), jnp.int32),
            pltpu.VMEM((EB,), jnp.int32),
            pltpu.VMEM((EB,), jnp.float32),
            pltpu.VMEM((SHARE,), jnp.float32),
            pltpu.VMEM((EB, 128), jnp.float32),
            pltpu.VMEM((EB, 128), jnp.float32),
            pltpu.VMEM_SHARED((ACC_ROWS, 128), jnp.float32),
            pltpu.VMEM_SHARED((ACC_ROWS,), jnp.float32),
            pltpu.SemaphoreType.DMA,
            pltpu.SemaphoreType.DMA,
        ],
    )


def _fused_layer_kernel(sa_ref, sb_ref, inva_ref, invb_ref, x_ref,
                        wla_ref, wlb_ref, wr_ref, b_ref, out_ref):
    ma = sa_ref[...] * inva_ref[...]
    mb = sb_ref[...] * invb_ref[...]
    acc = jnp.dot(ma, wla_ref[...], preferred_element_type=jnp.float32)
    acc += jnp.dot(mb, wlb_ref[...], preferred_element_type=jnp.float32)
    acc += jnp.dot(x_ref[...], wr_ref[...], preferred_element_type=jnp.float32)
    acc += b_ref[...]
    out_ref[...] = jnp.maximum(acc, 0.0)


def _fused_final_kernel(sa_ref, sb_ref, inva_ref, invb_ref, x_ref,
                        wla_ref, wlb_ref, wr_ref, b_ref,
                        wlin_ref, blin_ref, out_ref):
    ma = sa_ref[...] * inva_ref[...]
    mb = sb_ref[...] * invb_ref[...]
    acc = jnp.dot(ma, wla_ref[...], preferred_element_type=jnp.float32)
    acc += jnp.dot(mb, wlb_ref[...], preferred_element_type=jnp.float32)
    acc += jnp.dot(x_ref[...], wr_ref[...], preferred_element_type=jnp.float32)
    acc += b_ref[...]
    h = jnp.maximum(acc, 0.0)
    out_ref[...] = jnp.dot(h, wlin_ref[...],
                           preferred_element_type=jnp.float32) + blin_ref[...]


def _row_spec():
    return pl.BlockSpec((ROW_BLOCK, 128), lambda i: (i, 0))


def _inv_spec():
    return pl.BlockSpec((ROW_BLOCK, 1), lambda i: (i, 0))


def _w_spec():
    return pl.BlockSpec((128, 128), lambda i: (0, 0))


def _b_spec():
    return pl.BlockSpec((1, 128), lambda i: (0, 0))


def _fused_layer(sa, sb, inva, invb, x, wla, wlb, wr, b):
    return pl.pallas_call(
        _fused_layer_kernel,
        grid=(N_PAD // ROW_BLOCK,),
        in_specs=[_row_spec(), _row_spec(), _inv_spec(), _inv_spec(),
                  _row_spec(), _w_spec(), _w_spec(), _w_spec(), _b_spec()],
        out_specs=_row_spec(),
        out_shape=jax.ShapeDtypeStruct((N_PAD, 128), jnp.float32),
    )(sa, sb, inva, invb, x, wla, wlb, wr, b)


def _fused_final(sa, sb, inva, invb, x, wla, wlb, wr, b, wlin, blin):
    return pl.pallas_call(
        _fused_final_kernel,
        grid=(N_PAD // ROW_BLOCK,),
        in_specs=[_row_spec(), _row_spec(), _inv_spec(), _inv_spec(),
                  _row_spec(), _w_spec(), _w_spec(), _w_spec(), _b_spec(),
                  _w_spec(), _b_spec()],
        out_specs=_row_spec(),
        out_shape=jax.ShapeDtypeStruct((N_PAD, 128), jnp.float32),
    )(sa, sb, inva, invb, x, wla, wlb, wr, b, wlin, blin)


def kernel(x_note, edge_index_to, edge_index_rev_to, Wl0_to, bl0_to, Wr0_to,
           Wl0_rev, bl0_rev, Wr0_rev, Wl1_to, bl1_to, Wr1_to, Wl1_rev,
           bl1_rev, Wr1_rev, W_lin, b_lin):
    pad_e = E_PAD - E
    src_to = jnp.concatenate([edge_index_to[0], jnp.zeros((pad_e,), jnp.int32)])
    dst_to = jnp.concatenate([edge_index_to[1],
                              jnp.full((pad_e,), 1 << 30, jnp.int32)])
    src_rv = jnp.concatenate([edge_index_rev_to[0],
                              jnp.zeros((pad_e,), jnp.int32)])
    dst_rv = jnp.concatenate([edge_index_rev_to[1],
                              jnp.full((pad_e,), 1 << 30, jnp.int32)])

    agg0 = _make_sc_agg(N)
    agg1 = _make_sc_agg(N_PAD)

    s0_to, cnt_to = agg0(x_note, src_to, dst_to)
    s0_rv, cnt_rv = agg0(x_note, src_rv, dst_rv)

    inv_to = (1.0 / jnp.clip(cnt_to, 1.0, None))[:, None]
    inv_rv = (1.0 / jnp.clip(cnt_rv, 1.0, None))[:, None]

    x_p = jnp.pad(x_note, ((0, N_PAD - N), (0, 0)))
    b0 = (bl0_to + bl0_rev)[None, :]
    h = _fused_layer(s0_to, s0_rv, inv_to, inv_rv, x_p,
                     Wl0_to, Wl0_rev, Wr0_to + Wr0_rev, b0)

    s1_to, _ = agg1(h, src_to, dst_to)
    s1_rv, _ = agg1(h, src_rv, dst_rv)

    b1 = (bl1_to + bl1_rev)[None, :]
    out = _fused_final(s1_to, s1_rv, inv_to, inv_rv, h,
                       Wl1_to, Wl1_rev, Wr1_to + Wr1_rev, b1,
                       W_lin, b_lin[None, :])
    return out[:N]


# speedup vs baseline: 1.9845x; 1.4778x over previous
import jax
import jax.numpy as jnp
from jax import lax
from jax.experimental import pallas as pl
from jax.experimental.pallas import tpu as pltpu
from jax.experimental.pallas import tpu_sc as plsc

N = 50000
E = 400000

NC = 2
NS = 16

N_PAD = 50176
PASS_ROWS = N_PAD // 2
SHARE = PASS_ROWS // NS
GARB = PASS_ROWS
ACC_ROWS = PASS_ROWS + 16

EB = 128
E_PER_TILE = 25088
E_PAD = NS * E_PER_TILE
NBLK = E_PER_TILE // EB

ROW_BLOCK = 1024


def _sc_agg_body(x2_ref, src_ref, dst_ref, agg0_ref, agg1_ref, cnt_ref,
                 src_a, src_b, dst_a, dst_b, gidx_a, gidx_b, dloc_a, dloc_b,
                 ones_v, cstage_v, rows_a, rows_b, acc_sh, cnt_sh,
                 sem_a, sem_b):
    c = lax.axis_index("c")
    t = lax.axis_index("s")

    def init16(i, _):
        off = pl.multiple_of(i * 16, 16)
        ones_v[pl.ds(off, 16)] = jnp.full((16,), 1.0, jnp.float32)
        return 0
    lax.fori_loop(0, EB // 16, init16, 0)

    for p in range(2):
        lo = p * PASS_ROWS

        def zrow(i, _):
            r = i // 4
            off = pl.multiple_of((i % 4) * 16, 16)
            rows_a[r, pl.ds(off, 16)] = jnp.zeros((16,), jnp.float32)
            return 0
        lax.fori_loop(0, EB * 4, zrow, 0)

        def zc(i, _):
            off = pl.multiple_of(i * 16, 16)
            cstage_v[pl.ds(off, 16)] = jnp.zeros((16,), jnp.float32)
            return 0
        lax.fori_loop(0, SHARE // 16, zc, 0)

        for z in range(SHARE // EB):
            pltpu.sync_copy(rows_a, acc_sh.at[pl.ds(t * SHARE + z * EB, EB)])
        pltpu.sync_copy(rows_a.at[pl.ds(0, SHARE % EB)],
                        acc_sh.at[pl.ds(t * SHARE + (SHARE // EB) * EB,
                                        SHARE % EB)])

        @pl.when(c == 0)
        def _():
            pltpu.sync_copy(cstage_v, cnt_sh.at[pl.ds(t * SHARE, SHARE)])
        plsc.subcore_barrier()

        def stage(b, sv, dv, gi, dl):
            base = pl.multiple_of(t * E_PER_TILE + b * EB, EB)
            pltpu.sync_copy(src_ref.at[pl.ds(base, EB)], sv)
            pltpu.sync_copy(dst_ref.at[pl.ds(base, EB)], dv)

            def lane(j, _):
                off = pl.multiple_of(j * 16, 16)
                svv = sv[pl.ds(off, 16)]
                dvv = dv[pl.ds(off, 16)]
                gi[pl.ds(off, 16)] = 2 * svv + c
                inside = (dvv >= lo) & (dvv < lo + PASS_ROWS)
                dl[pl.ds(off, 16)] = jnp.where(inside, dvv - lo, GARB)
                return 0
            lax.fori_loop(0, EB // 16, lane, 0)

        def scatter(rv, dl):
            pltpu.sync_copy(rv, acc_sh.at[dl], add=True)

            @pl.when(c == 0)
            def _():
                pltpu.sync_copy(ones_v, cnt_sh.at[dl], add=True)

        stage(0, src_a, dst_a, gidx_a, dloc_a)
        pltpu.async_copy(x2_ref.at[gidx_a], rows_a, sem_a)

        def pair(g, _):
            stage(2 * g + 1, src_b, dst_b, gidx_b, dloc_b)
            pltpu.async_copy(x2_ref.at[gidx_b], rows_b, sem_b)
            pltpu.make_async_copy(x2_ref.at[gidx_a], rows_a, sem_a).wait()
            scatter(rows_a, dloc_a)

            @pl.when(g < NBLK // 2 - 1)
            def _():
                stage(2 * g + 2, src_a, dst_a, gidx_a, dloc_a)
                pltpu.async_copy(x2_ref.at[gidx_a], rows_a, sem_a)

            pltpu.make_async_copy(x2_ref.at[gidx_b], rows_b, sem_b).wait()
            scatter(rows_b, dloc_b)
            return 0
        lax.fori_loop(0, NBLK // 2, pair, 0)
        plsc.subcore_barrier()

        row0 = lo + t * SHARE

        @pl.when(c == 0)
        def _():
            pltpu.sync_copy(acc_sh.at[pl.ds(t * SHARE, SHARE)],
                            agg0_ref.at[pl.ds(row0, SHARE)])
            pltpu.sync_copy(cnt_sh.at[pl.ds(t * SHARE, SHARE)], cstage_v)
            pltpu.sync_copy(cstage_v, cnt_ref.at[pl.ds(row0, SHARE)])

        @pl.when(c == 1)
        def _():
            pltpu.sync_copy(acc_sh.at[pl.ds(t * SHARE, SHARE)],
                            agg1_ref.at[pl.ds(row0, SHARE)])


def _make_sc_agg(table_rows):
    mesh = plsc.VectorSubcoreMesh(core_axis_name="c", subcore_axis_name="s",
                                  num_cores=NC, num_subcores=NS)
    return pl.kernel(
        _sc_agg_body,
        out_type=(jax.ShapeDtypeStruct((N_PAD, 64), jnp.float32),
                  jax.ShapeDtypeStruct((N_PAD, 64), jnp.float32),
                  jax.ShapeDtypeStruct((N_PAD,), jnp.float32)),
        mesh=mesh,
        compiler_params=pltpu.CompilerParams(use_tc_tiling_on_sc=False),
        scratch_types=[
            pltpu.VMEM((EB,), jnp.int32),
            pltpu.VMEM((EB,), jnp.int32),
            pltpu.VMEM((EB,), jnp.int32),
            pltpu.VMEM((EB,), jnp.int32),
            pltpu.VMEM((EB,), jnp.int32),
            pltpu.VMEM((EB,), jnp.int32),
            pltpu.VMEM((EB,), jnp.int32),
            pltpu.VMEM((EB,), jnp.int32),
            pltpu.VMEM((EB,), jnp.float32),
            pltpu.VMEM((SHARE,), jnp.float32),
            pltpu.VMEM((EB, 64), jnp.float32),
            pltpu.VMEM((EB, 64), jnp.float32),
            pltpu.VMEM_SHARED((ACC_ROWS, 64), jnp.float32),
            pltpu.VMEM_SHARED((ACC_ROWS,), jnp.float32),
            pltpu.SemaphoreType.DMA,
            pltpu.SemaphoreType.DMA,
        ],
    )


def _fused_layer_kernel(sa0_ref, sa1_ref, sb0_ref, sb1_ref,
                        inva_ref, invb_ref, x_ref,
                        wla_ref, wlb_ref, wr_ref, b_ref, out_ref):
    acc = jnp.dot(sa0_ref[...] * inva_ref[...], wla_ref[0:64, :],
                  preferred_element_type=jnp.float32)
    acc += jnp.dot(sa1_ref[...] * inva_ref[...], wla_ref[64:128, :],
                   preferred_element_type=jnp.float32)
    acc += jnp.dot(sb0_ref[...] * invb_ref[...], wlb_ref[0:64, :],
                   preferred_element_type=jnp.float32)
    acc += jnp.dot(sb1_ref[...] * invb_ref[...], wlb_ref[64:128, :],
                   preferred_element_type=jnp.float32)
    acc += jnp.dot(x_ref[...], wr_ref[...], preferred_element_type=jnp.float32)
    acc += b_ref[...]
    out_ref[...] = jnp.maximum(acc, 0.0)


def _fused_final_kernel(sa0_ref, sa1_ref, sb0_ref, sb1_ref,
                        inva_ref, invb_ref, x_ref,
                        wla_ref, wlb_ref, wr_ref, b_ref,
                        wlin_ref, blin_ref, out_ref):
    acc = jnp.dot(sa0_ref[...] * inva_ref[...], wla_ref[0:64, :],
                  preferred_element_type=jnp.float32)
    acc += jnp.dot(sa1_ref[...] * inva_ref[...], wla_ref[64:128, :],
                   preferred_element_type=jnp.float32)
    acc += jnp.dot(sb0_ref[...] * invb_ref[...], wlb_ref[0:64, :],
                   preferred_element_type=jnp.float32)
    acc += jnp.dot(sb1_ref[...] * invb_ref[...], wlb_ref[64:128, :],
                   preferred_element_type=jnp.float32)
    acc += jnp.dot(x_ref[...], wr_ref[...], preferred_element_type=jnp.float32)
    acc += b_ref[...]
    h = jnp.maximum(acc, 0.0)
    out_ref[...] = jnp.dot(h, wlin_ref[...],
                           preferred_element_type=jnp.float32) + blin_ref[...]


def _row_spec():
    return pl.BlockSpec((ROW_BLOCK, 128), lambda i: (i, 0))


def _half_spec():
    return pl.BlockSpec((ROW_BLOCK, 64), lambda i: (i, 0))


def _inv_spec():
    return pl.BlockSpec((ROW_BLOCK, 1), lambda i: (i, 0))


def _w_spec():
    return pl.BlockSpec((128, 128), lambda i: (0, 0))


def _b_spec():
    return pl.BlockSpec((1, 128), lambda i: (0, 0))


def _fused_layer(sa0, sa1, sb0, sb1, inva, invb, x, wla, wlb, wr, b):
    return pl.pallas_call(
        _fused_layer_kernel,
        grid=(N_PAD // ROW_BLOCK,),
        in_specs=[_half_spec(), _half_spec(), _half_spec(), _half_spec(),
                  _inv_spec(), _inv_spec(),
                  _row_spec(), _w_spec(), _w_spec(), _w_spec(), _b_spec()],
        out_specs=_row_spec(),
        out_shape=jax.ShapeDtypeStruct((N_PAD, 128), jnp.float32),
    )(sa0, sa1, sb0, sb1, inva, invb, x, wla, wlb, wr, b)


def _fused_final(sa0, sa1, sb0, sb1, inva, invb, x, wla, wlb, wr, b,
                 wlin, blin):
    return pl.pallas_call(
        _fused_final_kernel,
        grid=(N_PAD // ROW_BLOCK,),
        in_specs=[_half_spec(), _half_spec(), _half_spec(), _half_spec(),
                  _inv_spec(), _inv_spec(),
                  _row_spec(), _w_spec(), _w_spec(), _w_spec(), _b_spec(),
                  _w_spec(), _b_spec()],
        out_specs=_row_spec(),
        out_shape=jax.ShapeDtypeStruct((N_PAD, 128), jnp.float32),
    )(sa0, sa1, sb0, sb1, inva, invb, x, wla, wlb, wr, b, wlin, blin)


def kernel(x_note, edge_index_to, edge_index_rev_to, Wl0_to, bl0_to, Wr0_to,
           Wl0_rev, bl0_rev, Wr0_rev, Wl1_to, bl1_to, Wr1_to, Wl1_rev,
           bl1_rev, Wr1_rev, W_lin, b_lin):
    pad_e = E_PAD - E
    src_to = jnp.concatenate([edge_index_to[0], jnp.zeros((pad_e,), jnp.int32)])
    dst_to = jnp.concatenate([edge_index_to[1],
                              jnp.full((pad_e,), 1 << 30, jnp.int32)])
    src_rv = jnp.concatenate([edge_index_rev_to[0],
                              jnp.zeros((pad_e,), jnp.int32)])
    dst_rv = jnp.concatenate([edge_index_rev_to[1],
                              jnp.full((pad_e,), 1 << 30, jnp.int32)])

    agg0 = _make_sc_agg(2 * N)
    agg1 = _make_sc_agg(2 * N_PAD)

    x2 = x_note.reshape(2 * N, 64)
    s0_to_h0, s0_to_h1, cnt_to = agg0(x2, src_to, dst_to)
    s0_rv_h0, s0_rv_h1, cnt_rv = agg0(x2, src_rv, dst_rv)

    inv_to = (1.0 / jnp.clip(cnt_to, 1.0, None))[:, None]
    inv_rv = (1.0 / jnp.clip(cnt_rv, 1.0, None))[:, None]

    x_p = jnp.pad(x_note, ((0, N_PAD - N), (0, 0)))
    b0 = (bl0_to + bl0_rev)[None, :]
    h = _fused_layer(s0_to_h0, s0_to_h1, s0_rv_h0, s0_rv_h1,
                     inv_to, inv_rv, x_p,
                     Wl0_to, Wl0_rev, Wr0_to + Wr0_rev, b0)

    h2 = h.reshape(2 * N_PAD, 64)
    s1_to_h0, s1_to_h1, _ = agg1(h2, src_to, dst_to)
    s1_rv_h0, s1_rv_h1, _ = agg1(h2, src_rv, dst_rv)

    b1 = (bl1_to + bl1_rev)[None, :]
    out = _fused_final(s1_to_h0, s1_to_h1, s1_rv_h0, s1_rv_h1,
                       inv_to, inv_rv, h,
                       Wl1_to, Wl1_rev, Wr1_to + Wr1_rev, b1,
                       W_lin, b_lin[None, :])
    return out[:N]


# speedup vs baseline: 2.4230x; 1.2210x over previous
import jax
import jax.numpy as jnp
from jax import lax
from jax.experimental import pallas as pl
from jax.experimental.pallas import tpu as pltpu
from jax.experimental.pallas import tpu_sc as plsc

N = 50000
E = 400000

NC = 2
NS = 16

N_PAD = 50176
PASS_ROWS = N_PAD // 2
SHARE = PASS_ROWS // NS
GARB = PASS_ROWS
ACC_ROWS = PASS_ROWS + 16

EB = 128
E_PER_TILE = 25088
E_PAD = NS * E_PER_TILE
NBLK = E_PER_TILE // EB

ROW_BLOCK = 1024


def _sc_agg_body(with_cnt, x2_ref, src_ref, dst_ref, agg0_ref, agg1_ref,
                 cnt_ref,
                 src_a, src_b, dst_a, dst_b, gidx_a, gidx_b, dloc_a, dloc_b,
                 ones_v, cstage_v, rows_a, rows_b, acc_sh, cnt_sh,
                 sem_a, sem_b):
    c = lax.axis_index("c")
    t = lax.axis_index("s")

    def init16(i, _):
        off = pl.multiple_of(i * 16, 16)
        ones_v[pl.ds(off, 16)] = jnp.full((16,), 1.0, jnp.float32)
        return 0
    lax.fori_loop(0, EB // 16, init16, 0)

    for p in range(2):
        lo = p * PASS_ROWS

        def zrow(i, _):
            r = i // 4
            off = pl.multiple_of((i % 4) * 16, 16)
            rows_a[r, pl.ds(off, 16)] = jnp.zeros((16,), jnp.float32)
            return 0
        lax.fori_loop(0, EB * 4, zrow, 0)

        def zc(i, _):
            off = pl.multiple_of(i * 16, 16)
            cstage_v[pl.ds(off, 16)] = jnp.zeros((16,), jnp.float32)
            return 0
        lax.fori_loop(0, SHARE // 16, zc, 0)

        for z in range(SHARE // EB):
            pltpu.sync_copy(rows_a, acc_sh.at[pl.ds(t * SHARE + z * EB, EB)])
        pltpu.sync_copy(rows_a.at[pl.ds(0, SHARE % EB)],
                        acc_sh.at[pl.ds(t * SHARE + (SHARE // EB) * EB,
                                        SHARE % EB)])

        if with_cnt:
            @pl.when(c == p)
            def _():
                pltpu.sync_copy(cstage_v, cnt_sh.at[pl.ds(t * SHARE, SHARE)])
        plsc.subcore_barrier()

        def stage(b, sv, dv, gi, dl):
            base = pl.multiple_of(t * E_PER_TILE + b * EB, EB)
            pltpu.sync_copy(src_ref.at[pl.ds(base, EB)], sv)
            pltpu.sync_copy(dst_ref.at[pl.ds(base, EB)], dv)

            def lane(j, _):
                off = pl.multiple_of(j * 16, 16)
                svv = sv[pl.ds(off, 16)]
                dvv = dv[pl.ds(off, 16)]
                gi[pl.ds(off, 16)] = 2 * svv + c
                inside = (dvv >= lo) & (dvv < lo + PASS_ROWS)
                dl[pl.ds(off, 16)] = jnp.where(inside, dvv - lo, GARB)
                return 0
            lax.fori_loop(0, EB // 16, lane, 0)

        def scatter(rv, dl):
            pltpu.sync_copy(rv, acc_sh.at[dl], add=True)
            if with_cnt:
                @pl.when(c == p)
                def _():
                    pltpu.sync_copy(ones_v, cnt_sh.at[dl], add=True)

        stage(0, src_a, dst_a, gidx_a, dloc_a)
        pltpu.async_copy(x2_ref.at[gidx_a], rows_a, sem_a)

        def pair(g, _):
            stage(2 * g + 1, src_b, dst_b, gidx_b, dloc_b)
            pltpu.async_copy(x2_ref.at[gidx_b], rows_b, sem_b)
            pltpu.make_async_copy(x2_ref.at[gidx_a], rows_a, sem_a).wait()
            scatter(rows_a, dloc_a)

            @pl.when(g < NBLK // 2 - 1)
            def _():
                stage(2 * g + 2, src_a, dst_a, gidx_a, dloc_a)
                pltpu.async_copy(x2_ref.at[gidx_a], rows_a, sem_a)

            pltpu.make_async_copy(x2_ref.at[gidx_b], rows_b, sem_b).wait()
            scatter(rows_b, dloc_b)
            return 0
        lax.fori_loop(0, NBLK // 2, pair, 0)
        plsc.subcore_barrier()

        row0 = lo + t * SHARE

        @pl.when(c == 0)
        def _():
            pltpu.sync_copy(acc_sh.at[pl.ds(t * SHARE, SHARE)],
                            agg0_ref.at[pl.ds(row0, SHARE)])

        @pl.when(c == 1)
        def _():
            pltpu.sync_copy(acc_sh.at[pl.ds(t * SHARE, SHARE)],
                            agg1_ref.at[pl.ds(row0, SHARE)])

        if with_cnt:
            @pl.when(c == p)
            def _():
                pltpu.sync_copy(cnt_sh.at[pl.ds(t * SHARE, SHARE)], cstage_v)
                pltpu.sync_copy(cstage_v, cnt_ref.at[pl.ds(row0, SHARE)])


def _make_sc_agg(with_cnt):
    import functools as _ft
    mesh = plsc.VectorSubcoreMesh(core_axis_name="c", subcore_axis_name="s",
                                  num_cores=NC, num_subcores=NS)
    return pl.kernel(
        _ft.partial(_sc_agg_body, with_cnt),
        out_type=(jax.ShapeDtypeStruct((N_PAD, 64), jnp.float32),
                  jax.ShapeDtypeStruct((N_PAD, 64), jnp.float32),
                  jax.ShapeDtypeStruct((N_PAD,), jnp.float32)),
        mesh=mesh,
        compiler_params=pltpu.CompilerParams(use_tc_tiling_on_sc=False),
        scratch_types=[
            pltpu.VMEM((EB,), jnp.int32),
            pltpu.VMEM((EB,), jnp.int32),
            pltpu.VMEM((EB,), jnp.int32),
            pltpu.VMEM((EB,), jnp.int32),
            pltpu.VMEM((EB,), jnp.int32),
            pltpu.VMEM((EB,), jnp.int32),
            pltpu.VMEM((EB,), jnp.int32),
            pltpu.VMEM((EB,), jnp.int32),
            pltpu.VMEM((EB,), jnp.float32),
            pltpu.VMEM((SHARE,), jnp.float32),
            pltpu.VMEM((EB, 64), jnp.float32),
            pltpu.VMEM((EB, 64), jnp.float32),
            pltpu.VMEM_SHARED((ACC_ROWS, 64), jnp.float32),
            pltpu.VMEM_SHARED((ACC_ROWS,), jnp.float32),
            pltpu.SemaphoreType.DMA,
            pltpu.SemaphoreType.DMA,
        ],
    )


def _fused_layer_kernel(sa0_ref, sa1_ref, sb0_ref, sb1_ref,
                        inva_ref, invb_ref, x_ref,
                        wla_ref, wlb_ref, wr_ref, b_ref, out_ref):
    acc = jnp.dot(sa0_ref[...] * inva_ref[...], wla_ref[0:64, :],
                  preferred_element_type=jnp.float32)
    acc += jnp.dot(sa1_ref[...] * inva_ref[...], wla_ref[64:128, :],
                   preferred_element_type=jnp.float32)
    acc += jnp.dot(sb0_ref[...] * invb_ref[...], wlb_ref[0:64, :],
                   preferred_element_type=jnp.float32)
    acc += jnp.dot(sb1_ref[...] * invb_ref[...], wlb_ref[64:128, :],
                   preferred_element_type=jnp.float32)
    acc += jnp.dot(x_ref[...], wr_ref[...], preferred_element_type=jnp.float32)
    acc += b_ref[...]
    out_ref[...] = jnp.maximum(acc, 0.0)


def _fused_final_kernel(sa0_ref, sa1_ref, sb0_ref, sb1_ref,
                        inva_ref, invb_ref, x_ref,
                        wla_ref, wlb_ref, wr_ref, b_ref,
                        wlin_ref, blin_ref, out_ref):
    acc = jnp.dot(sa0_ref[...] * inva_ref[...], wla_ref[0:64, :],
                  preferred_element_type=jnp.float32)
    acc += jnp.dot(sa1_ref[...] * inva_ref[...], wla_ref[64:128, :],
                   preferred_element_type=jnp.float32)
    acc += jnp.dot(sb0_ref[...] * invb_ref[...], wlb_ref[0:64, :],
                   preferred_element_type=jnp.float32)
    acc += jnp.dot(sb1_ref[...] * invb_ref[...], wlb_ref[64:128, :],
                   preferred_element_type=jnp.float32)
    acc += jnp.dot(x_ref[...], wr_ref[...], preferred_element_type=jnp.float32)
    acc += b_ref[...]
    h = jnp.maximum(acc, 0.0)
    out_ref[...] = jnp.dot(h, wlin_ref[...],
                           preferred_element_type=jnp.float32) + blin_ref[...]


def _row_spec():
    return pl.BlockSpec((ROW_BLOCK, 128), lambda i: (i, 0))


def _half_spec():
    return pl.BlockSpec((ROW_BLOCK, 64), lambda i: (i, 0))


def _inv_spec():
    return pl.BlockSpec((ROW_BLOCK, 1), lambda i: (i, 0))


def _w_spec():
    return pl.BlockSpec((128, 128), lambda i: (0, 0))


def _b_spec():
    return pl.BlockSpec((1, 128), lambda i: (0, 0))


def _fused_layer(sa0, sa1, sb0, sb1, inva, invb, x, wla, wlb, wr, b):
    return pl.pallas_call(
        _fused_layer_kernel,
        grid=(N_PAD // ROW_BLOCK,),
        in_specs=[_half_spec(), _half_spec(), _half_spec(), _half_spec(),
                  _inv_spec(), _inv_spec(),
                  _row_spec(), _w_spec(), _w_spec(), _w_spec(), _b_spec()],
        out_specs=_row_spec(),
        out_shape=jax.ShapeDtypeStruct((N_PAD, 128), jnp.float32),
    )(sa0, sa1, sb0, sb1, inva, invb, x, wla, wlb, wr, b)


def _fused_final(sa0, sa1, sb0, sb1, inva, invb, x, wla, wlb, wr, b,
                 wlin, blin):
    return pl.pallas_call(
        _fused_final_kernel,
        grid=(N_PAD // ROW_BLOCK,),
        in_specs=[_half_spec(), _half_spec(), _half_spec(), _half_spec(),
                  _inv_spec(), _inv_spec(),
                  _row_spec(), _w_spec(), _w_spec(), _w_spec(), _b_spec(),
                  _w_spec(), _b_spec()],
        out_specs=_row_spec(),
        out_shape=jax.ShapeDtypeStruct((N_PAD, 128), jnp.float32),
    )(sa0, sa1, sb0, sb1, inva, invb, x, wla, wlb, wr, b, wlin, blin)


def kernel(x_note, edge_index_to, edge_index_rev_to, Wl0_to, bl0_to, Wr0_to,
           Wl0_rev, bl0_rev, Wr0_rev, Wl1_to, bl1_to, Wr1_to, Wl1_rev,
           bl1_rev, Wr1_rev, W_lin, b_lin):
    pad_e = E_PAD - E
    src_to = jnp.concatenate([edge_index_to[0], jnp.zeros((pad_e,), jnp.int32)])
    dst_to = jnp.concatenate([edge_index_to[1],
                              jnp.full((pad_e,), 1 << 30, jnp.int32)])
    src_rv = jnp.concatenate([edge_index_rev_to[0],
                              jnp.zeros((pad_e,), jnp.int32)])
    dst_rv = jnp.concatenate([edge_index_rev_to[1],
                              jnp.full((pad_e,), 1 << 30, jnp.int32)])

    agg0 = _make_sc_agg(True)
    agg1 = _make_sc_agg(False)

    x2 = x_note.reshape(2 * N, 64)
    s0_to_h0, s0_to_h1, cnt_to = agg0(x2, src_to, dst_to)
    s0_rv_h0, s0_rv_h1, cnt_rv = agg0(x2, src_rv, dst_rv)

    inv_to = (1.0 / jnp.clip(cnt_to, 1.0, None))[:, None]
    inv_rv = (1.0 / jnp.clip(cnt_rv, 1.0, None))[:, None]

    x_p = jnp.pad(x_note, ((0, N_PAD - N), (0, 0)))
    b0 = (bl0_to + bl0_rev)[None, :]
    h = _fused_layer(s0_to_h0, s0_to_h1, s0_rv_h0, s0_rv_h1,
                     inv_to, inv_rv, x_p,
                     Wl0_to, Wl0_rev, Wr0_to + Wr0_rev, b0)

    h2 = h.reshape(2 * N_PAD, 64)
    s1_to_h0, s1_to_h1, _ = agg1(h2, src_to, dst_to)
    s1_rv_h0, s1_rv_h1, _ = agg1(h2, src_rv, dst_rv)

    b1 = (bl1_to + bl1_rev)[None, :]
    out = _fused_final(s1_to_h0, s1_to_h1, s1_rv_h0, s1_rv_h1,
                       inv_to, inv_rv, h,
                       Wl1_to, Wl1_rev, Wr1_to + Wr1_rev, b1,
                       W_lin, b_lin[None, :])
    return out[:N]


# speedup vs baseline: 2.4503x; 1.0113x over previous
import jax
import jax.numpy as jnp
from jax import lax
from jax.experimental import pallas as pl
from jax.experimental.pallas import tpu as pltpu
from jax.experimental.pallas import tpu_sc as plsc

N = 50000
E = 400000

NC = 2
NS = 16

N_PAD = 50176
PASS_ROWS = N_PAD // 2
SHARE = PASS_ROWS // NS
GARB = PASS_ROWS
ACC_ROWS = PASS_ROWS + 16

EB = 128
E_PER_TILE = 25088
E_PAD = NS * E_PER_TILE
NBLK = E_PER_TILE // EB
SB = 4

ROW_BLOCK = 1024


def _sc_agg_body(with_cnt, x2_ref, src_ref, dst_ref, agg0_ref, agg1_ref,
                 cnt_ref,
                 src_big, dst_big, gidx_a, gidx_b, dloc_a, dloc_b,
                 ones_v, cstage_v, rows_a, rows_b, acc_sh, cnt_sh,
                 sem_a, sem_b):
    c = lax.axis_index("c")
    t = lax.axis_index("s")

    def init16(i, _):
        off = pl.multiple_of(i * 16, 16)
        ones_v[pl.ds(off, 16)] = jnp.full((16,), 1.0, jnp.float32)
        return 0
    lax.fori_loop(0, EB // 16, init16, 0)

    for p in range(2):
        lo = p * PASS_ROWS

        def zrow(i, _):
            r = i // 4
            off = pl.multiple_of((i % 4) * 16, 16)
            rows_a[r, pl.ds(off, 16)] = jnp.zeros((16,), jnp.float32)
            return 0
        lax.fori_loop(0, EB * 4, zrow, 0)

        def zc(i, _):
            off = pl.multiple_of(i * 16, 16)
            cstage_v[pl.ds(off, 16)] = jnp.zeros((16,), jnp.float32)
            return 0
        lax.fori_loop(0, SHARE // 16, zc, 0)

        for z in range(SHARE // EB):
            pltpu.sync_copy(rows_a, acc_sh.at[pl.ds(t * SHARE + z * EB, EB)])
        pltpu.sync_copy(rows_a.at[pl.ds(0, SHARE % EB)],
                        acc_sh.at[pl.ds(t * SHARE + (SHARE // EB) * EB,
                                        SHARE % EB)])

        if with_cnt:
            @pl.when(c == p)
            def _():
                pltpu.sync_copy(cstage_v, cnt_sh.at[pl.ds(t * SHARE, SHARE)])
        plsc.subcore_barrier()

        def stage_group(k):
            base = pl.multiple_of(t * E_PER_TILE + k * SB * EB, EB)
            pltpu.sync_copy(src_ref.at[pl.ds(base, SB * EB)], src_big)
            pltpu.sync_copy(dst_ref.at[pl.ds(base, SB * EB)], dst_big)

        def build(b, gi, dl):
            wo = (b % SB) * EB

            def lane(j, _):
                off = pl.multiple_of(wo + j * 16, 16)
                svv = src_big[pl.ds(off, 16)]
                dvv = dst_big[pl.ds(off, 16)]
                o2 = pl.multiple_of(j * 16, 16)
                gi[pl.ds(o2, 16)] = 2 * svv + c
                inside = (dvv >= lo) & (dvv < lo + PASS_ROWS)
                dl[pl.ds(o2, 16)] = jnp.where(inside, dvv - lo, GARB)
                return 0
            lax.fori_loop(0, EB // 16, lane, 0)

        def scatter(rv, dl):
            pltpu.sync_copy(rv, acc_sh.at[dl], add=True)
            if with_cnt:
                @pl.when(c == p)
                def _():
                    pltpu.sync_copy(ones_v, cnt_sh.at[dl], add=True)

        stage_group(0)
        build(0, gidx_a, dloc_a)
        pltpu.async_copy(x2_ref.at[gidx_a], rows_a, sem_a)

        def pair(g, _):
            build(2 * g + 1, gidx_b, dloc_b)
            pltpu.async_copy(x2_ref.at[gidx_b], rows_b, sem_b)
            pltpu.make_async_copy(x2_ref.at[gidx_a], rows_a, sem_a).wait()
            scatter(rows_a, dloc_a)

            @pl.when((g % 2 == 1) & (g < NBLK // 2 - 1))
            def _():
                stage_group((g + 1) // 2)

            @pl.when(g < NBLK // 2 - 1)
            def _():
                build(2 * g + 2, gidx_a, dloc_a)
                pltpu.async_copy(x2_ref.at[gidx_a], rows_a, sem_a)

            pltpu.make_async_copy(x2_ref.at[gidx_b], rows_b, sem_b).wait()
            scatter(rows_b, dloc_b)
            return 0
        lax.fori_loop(0, NBLK // 2, pair, 0)
        plsc.subcore_barrier()

        row0 = lo + t * SHARE

        @pl.when(c == 0)
        def _():
            pltpu.sync_copy(acc_sh.at[pl.ds(t * SHARE, SHARE)],
                            agg0_ref.at[pl.ds(row0, SHARE)])

        @pl.when(c == 1)
        def _():
            pltpu.sync_copy(acc_sh.at[pl.ds(t * SHARE, SHARE)],
                            agg1_ref.at[pl.ds(row0, SHARE)])

        if with_cnt:
            @pl.when(c == p)
            def _():
                pltpu.sync_copy(cnt_sh.at[pl.ds(t * SHARE, SHARE)], cstage_v)
                pltpu.sync_copy(cstage_v, cnt_ref.at[pl.ds(row0, SHARE)])


def _make_sc_agg(with_cnt):
    import functools as _ft
    mesh = plsc.VectorSubcoreMesh(core_axis_name="c", subcore_axis_name="s",
                                  num_cores=NC, num_subcores=NS)
    return pl.kernel(
        _ft.partial(_sc_agg_body, with_cnt),
        out_type=(jax.ShapeDtypeStruct((N_PAD, 64), jnp.float32),
                  jax.ShapeDtypeStruct((N_PAD, 64), jnp.float32),
                  jax.ShapeDtypeStruct((N_PAD,), jnp.float32)),
        mesh=mesh,
        compiler_params=pltpu.CompilerParams(use_tc_tiling_on_sc=False),
        scratch_types=[
            pltpu.VMEM((SB * EB,), jnp.int32),
            pltpu.VMEM((SB * EB,), jnp.int32),
            pltpu.VMEM((EB,), jnp.int32),
            pltpu.VMEM((EB,), jnp.int32),
            pltpu.VMEM((EB,), jnp.int32),
            pltpu.VMEM((EB,), jnp.int32),
            pltpu.VMEM((EB,), jnp.float32),
            pltpu.VMEM((SHARE,), jnp.float32),
            pltpu.VMEM((EB, 64), jnp.float32),
            pltpu.VMEM((EB, 64), jnp.float32),
            pltpu.VMEM_SHARED((ACC_ROWS, 64), jnp.float32),
            pltpu.VMEM_SHARED((ACC_ROWS,), jnp.float32),
            pltpu.SemaphoreType.DMA,
            pltpu.SemaphoreType.DMA,
        ],
    )


def _fused_layer_kernel(sa0_ref, sa1_ref, sb0_ref, sb1_ref,
                        inva_ref, invb_ref, x_ref,
                        wla_ref, wlb_ref, wr_ref, b_ref, out_ref):
    acc = jnp.dot(sa0_ref[...] * inva_ref[...], wla_ref[0:64, :],
                  preferred_element_type=jnp.float32)
    acc += jnp.dot(sa1_ref[...] * inva_ref[...], wla_ref[64:128, :],
                   preferred_element_type=jnp.float32)
    acc += jnp.dot(sb0_ref[...] * invb_ref[...], wlb_ref[0:64, :],
                   preferred_element_type=jnp.float32)
    acc += jnp.dot(sb1_ref[...] * invb_ref[...], wlb_ref[64:128, :],
                   preferred_element_type=jnp.float32)
    acc += jnp.dot(x_ref[...], wr_ref[...], preferred_element_type=jnp.float32)
    acc += b_ref[...]
    out_ref[...] = jnp.maximum(acc, 0.0)


def _fused_final_kernel(sa0_ref, sa1_ref, sb0_ref, sb1_ref,
                        inva_ref, invb_ref, x_ref,
                        wla_ref, wlb_ref, wr_ref, b_ref,
                        wlin_ref, blin_ref, out_ref):
    acc = jnp.dot(sa0_ref[...] * inva_ref[...], wla_ref[0:64, :],
                  preferred_element_type=jnp.float32)
    acc += jnp.dot(sa1_ref[...] * inva_ref[...], wla_ref[64:128, :],
                   preferred_element_type=jnp.float32)
    acc += jnp.dot(sb0_ref[...] * invb_ref[...], wlb_ref[0:64, :],
                   preferred_element_type=jnp.float32)
    acc += jnp.dot(sb1_ref[...] * invb_ref[...], wlb_ref[64:128, :],
                   preferred_element_type=jnp.float32)
    acc += jnp.dot(x_ref[...], wr_ref[...], preferred_element_type=jnp.float32)
    acc += b_ref[...]
    h = jnp.maximum(acc, 0.0)
    out_ref[...] = jnp.dot(h, wlin_ref[...],
                           preferred_element_type=jnp.float32) + blin_ref[...]


def _row_spec():
    return pl.BlockSpec((ROW_BLOCK, 128), lambda i: (i, 0))


def _half_spec():
    return pl.BlockSpec((ROW_BLOCK, 64), lambda i: (i, 0))


def _inv_spec():
    return pl.BlockSpec((ROW_BLOCK, 1), lambda i: (i, 0))


def _w_spec():
    return pl.BlockSpec((128, 128), lambda i: (0, 0))


def _b_spec():
    return pl.BlockSpec((1, 128), lambda i: (0, 0))


def _fused_layer(sa0, sa1, sb0, sb1, inva, invb, x, wla, wlb, wr, b):
    return pl.pallas_call(
        _fused_layer_kernel,
        grid=(N_PAD // ROW_BLOCK,),
        in_specs=[_half_spec(), _half_spec(), _half_spec(), _half_spec(),
                  _inv_spec(), _inv_spec(),
                  _row_spec(), _w_spec(), _w_spec(), _w_spec(), _b_spec()],
        out_specs=_row_spec(),
        out_shape=jax.ShapeDtypeStruct((N_PAD, 128), jnp.float32),
    )(sa0, sa1, sb0, sb1, inva, invb, x, wla, wlb, wr, b)


def _fused_final(sa0, sa1, sb0, sb1, inva, invb, x, wla, wlb, wr, b,
                 wlin, blin):
    return pl.pallas_call(
        _fused_final_kernel,
        grid=(N_PAD // ROW_BLOCK,),
        in_specs=[_half_spec(), _half_spec(), _half_spec(), _half_spec(),
                  _inv_spec(), _inv_spec(),
                  _row_spec(), _w_spec(), _w_spec(), _w_spec(), _b_spec(),
                  _w_spec(), _b_spec()],
        out_specs=_row_spec(),
        out_shape=jax.ShapeDtypeStruct((N_PAD, 128), jnp.float32),
    )(sa0, sa1, sb0, sb1, inva, invb, x, wla, wlb, wr, b, wlin, blin)


def kernel(x_note, edge_index_to, edge_index_rev_to, Wl0_to, bl0_to, Wr0_to,
           Wl0_rev, bl0_rev, Wr0_rev, Wl1_to, bl1_to, Wr1_to, Wl1_rev,
           bl1_rev, Wr1_rev, W_lin, b_lin):
    pad_e = E_PAD - E
    src_to = jnp.concatenate([edge_index_to[0], jnp.zeros((pad_e,), jnp.int32)])
    dst_to = jnp.concatenate([edge_index_to[1],
                              jnp.full((pad_e,), 1 << 30, jnp.int32)])
    src_rv = jnp.concatenate([edge_index_rev_to[0],
                              jnp.zeros((pad_e,), jnp.int32)])
    dst_rv = jnp.concatenate([edge_index_rev_to[1],
                              jnp.full((pad_e,), 1 << 30, jnp.int32)])

    agg0 = _make_sc_agg(True)
    agg1 = _make_sc_agg(False)

    x2 = x_note.reshape(2 * N, 64)
    s0_to_h0, s0_to_h1, cnt_to = agg0(x2, src_to, dst_to)
    s0_rv_h0, s0_rv_h1, cnt_rv = agg0(x2, src_rv, dst_rv)

    inv_to = (1.0 / jnp.clip(cnt_to, 1.0, None))[:, None]
    inv_rv = (1.0 / jnp.clip(cnt_rv, 1.0, None))[:, None]

    x_p = jnp.pad(x_note, ((0, N_PAD - N), (0, 0)))
    b0 = (bl0_to + bl0_rev)[None, :]
    h = _fused_layer(s0_to_h0, s0_to_h1, s0_rv_h0, s0_rv_h1,
                     inv_to, inv_rv, x_p,
                     Wl0_to, Wl0_rev, Wr0_to + Wr0_rev, b0)

    h2 = h.reshape(2 * N_PAD, 64)
    s1_to_h0, s1_to_h1, _ = agg1(h2, src_to, dst_to)
    s1_rv_h0, s1_rv_h1, _ = agg1(h2, src_rv, dst_rv)

    b1 = (bl1_to + bl1_rev)[None, :]
    out = _fused_final(s1_to_h0, s1_to_h1, s1_rv_h0, s1_rv_h1,
                       inv_to, inv_rv, h,
                       Wl1_to, Wl1_rev, Wr1_to + Wr1_rev, b1,
                       W_lin, b_lin[None, :])
    return out[:N]


# speedup vs baseline: 2.4553x; 1.0020x over previous
import jax
import jax.numpy as jnp
from jax import lax
from jax.experimental import pallas as pl
from jax.experimental.pallas import tpu as pltpu
from jax.experimental.pallas import tpu_sc as plsc

N = 50000
E = 400000

NC = 2
NS = 16

N_PAD = 50176
PASS_ROWS = N_PAD // 2
SHARE = PASS_ROWS // NS
GARB = PASS_ROWS
ACC_ROWS = PASS_ROWS + 16

EB = 128
E_PER_TILE = 25088
E_PAD = NS * E_PER_TILE
NBLK = E_PER_TILE // EB
SB = 4

ROW_BLOCK = 1024


def _sc_agg_body(with_cnt, x2_ref, src_ref, dst_ref, agg0_ref, agg1_ref,
                 cnt_ref,
                 src_big, dst_big, gidx_a, gidx_b, dloc_a, dloc_b,
                 ones_v, cstage_v, rows_a, rows_b, acc_sh, cnt_sh,
                 sem_a, sem_b, sem_sa, sem_sb, sem_ca, sem_cb):
    c = lax.axis_index("c")
    t = lax.axis_index("s")

    def init16(i, _):
        off = pl.multiple_of(i * 16, 16)
        ones_v[pl.ds(off, 16)] = jnp.full((16,), 1.0, jnp.float32)
        return 0
    lax.fori_loop(0, EB // 16, init16, 0)

    for p in range(2):
        lo = p * PASS_ROWS

        def zrow(i, _):
            r = i // 4
            off = pl.multiple_of((i % 4) * 16, 16)
            rows_a[r, pl.ds(off, 16)] = jnp.zeros((16,), jnp.float32)
            return 0
        lax.fori_loop(0, EB * 4, zrow, 0)

        def zc(i, _):
            off = pl.multiple_of(i * 16, 16)
            cstage_v[pl.ds(off, 16)] = jnp.zeros((16,), jnp.float32)
            return 0
        lax.fori_loop(0, SHARE // 16, zc, 0)

        for z in range(SHARE // EB):
            pltpu.sync_copy(rows_a, acc_sh.at[pl.ds(t * SHARE + z * EB, EB)])
        pltpu.sync_copy(rows_a.at[pl.ds(0, SHARE % EB)],
                        acc_sh.at[pl.ds(t * SHARE + (SHARE // EB) * EB,
                                        SHARE % EB)])

        if with_cnt:
            @pl.when(c == p)
            def _():
                pltpu.sync_copy(cstage_v, cnt_sh.at[pl.ds(t * SHARE, SHARE)])
        plsc.subcore_barrier()

        def stage_group(k):
            base = pl.multiple_of(t * E_PER_TILE + k * SB * EB, EB)
            pltpu.sync_copy(src_ref.at[pl.ds(base, SB * EB)], src_big)
            pltpu.sync_copy(dst_ref.at[pl.ds(base, SB * EB)], dst_big)

        def build(b, gi, dl):
            wo = (b % SB) * EB

            def lane(j, _):
                off = pl.multiple_of(wo + j * 16, 16)
                svv = src_big[pl.ds(off, 16)]
                dvv = dst_big[pl.ds(off, 16)]
                o2 = pl.multiple_of(j * 16, 16)
                gi[pl.ds(o2, 16)] = 2 * svv + c
                inside = (dvv >= lo) & (dvv < lo + PASS_ROWS)
                dl[pl.ds(o2, 16)] = jnp.where(inside, dvv - lo, GARB)
                return 0
            lax.fori_loop(0, EB // 16, lane, 0)

        def scatter_fire(rv, dl, sem_s, sem_c):
            pltpu.async_copy(rv, acc_sh.at[dl], sem_s, add=True)
            if with_cnt:
                @pl.when(c == p)
                def _():
                    pltpu.async_copy(ones_v, cnt_sh.at[dl], sem_c, add=True)

        def scatter_wait(rv, dl, sem_s, sem_c):
            pltpu.make_async_copy(rv, acc_sh.at[dl], sem_s).wait()
            if with_cnt:
                @pl.when(c == p)
                def _():
                    pltpu.make_async_copy(ones_v, cnt_sh.at[dl], sem_c).wait()

        stage_group(0)
        build(0, gidx_a, dloc_a)
        pltpu.async_copy(x2_ref.at[gidx_a], rows_a, sem_a)

        def pair(g, _):
            @pl.when(g > 0)
            def _():
                scatter_wait(rows_b, dloc_b, sem_sb, sem_cb)

            build(2 * g + 1, gidx_b, dloc_b)
            pltpu.async_copy(x2_ref.at[gidx_b], rows_b, sem_b)
            pltpu.make_async_copy(x2_ref.at[gidx_a], rows_a, sem_a).wait()
            scatter_fire(rows_a, dloc_a, sem_sa, sem_ca)

            @pl.when((g % 2 == 1) & (g < NBLK // 2 - 1))
            def _():
                stage_group((g + 1) // 2)

            @pl.when(g < NBLK // 2 - 1)
            def _():
                scatter_wait(rows_a, dloc_a, sem_sa, sem_ca)
                build(2 * g + 2, gidx_a, dloc_a)
                pltpu.async_copy(x2_ref.at[gidx_a], rows_a, sem_a)

            @pl.when(g == NBLK // 2 - 1)
            def _():
                scatter_wait(rows_a, dloc_a, sem_sa, sem_ca)

            pltpu.make_async_copy(x2_ref.at[gidx_b], rows_b, sem_b).wait()
            scatter_fire(rows_b, dloc_b, sem_sb, sem_cb)
            return 0
        lax.fori_loop(0, NBLK // 2, pair, 0)
        scatter_wait(rows_b, dloc_b, sem_sb, sem_cb)
        plsc.subcore_barrier()

        row0 = lo + t * SHARE

        @pl.when(c == 0)
        def _():
            pltpu.sync_copy(acc_sh.at[pl.ds(t * SHARE, SHARE)],
                            agg0_ref.at[pl.ds(row0, SHARE)])

        @pl.when(c == 1)
        def _():
            pltpu.sync_copy(acc_sh.at[pl.ds(t * SHARE, SHARE)],
                            agg1_ref.at[pl.ds(row0, SHARE)])

        if with_cnt:
            @pl.when(c == p)
            def _():
                pltpu.sync_copy(cnt_sh.at[pl.ds(t * SHARE, SHARE)], cstage_v)
                pltpu.sync_copy(cstage_v, cnt_ref.at[pl.ds(row0, SHARE)])


def _make_sc_agg(with_cnt):
    import functools as _ft
    mesh = plsc.VectorSubcoreMesh(core_axis_name="c", subcore_axis_name="s",
                                  num_cores=NC, num_subcores=NS)
    return pl.kernel(
        _ft.partial(_sc_agg_body, with_cnt),
        out_type=(jax.ShapeDtypeStruct((N_PAD, 64), jnp.float32),
                  jax.ShapeDtypeStruct((N_PAD, 64), jnp.float32),
                  jax.ShapeDtypeStruct((N_PAD,), jnp.float32)),
        mesh=mesh,
        compiler_params=pltpu.CompilerParams(use_tc_tiling_on_sc=False),
        scratch_types=[
            pltpu.VMEM((SB * EB,), jnp.int32),
            pltpu.VMEM((SB * EB,), jnp.int32),
            pltpu.VMEM((EB,), jnp.int32),
            pltpu.VMEM((EB,), jnp.int32),
            pltpu.VMEM((EB,), jnp.int32),
            pltpu.VMEM((EB,), jnp.int32),
            pltpu.VMEM((EB,), jnp.float32),
            pltpu.VMEM((SHARE,), jnp.float32),
            pltpu.VMEM((EB, 64), jnp.float32),
            pltpu.VMEM((EB, 64), jnp.float32),
            pltpu.VMEM_SHARED((ACC_ROWS, 64), jnp.float32),
            pltpu.VMEM_SHARED((ACC_ROWS,), jnp.float32),
            pltpu.SemaphoreType.DMA,
            pltpu.SemaphoreType.DMA,
            pltpu.SemaphoreType.DMA,
            pltpu.SemaphoreType.DMA,
            pltpu.SemaphoreType.DMA,
            pltpu.SemaphoreType.DMA,
        ],
    )


def _fused_layer_kernel(sa0_ref, sa1_ref, sb0_ref, sb1_ref,
                        inva_ref, invb_ref, x_ref,
                        wla_ref, wlb_ref, wr_ref, b_ref, out_ref):
    acc = jnp.dot(sa0_ref[...] * inva_ref[...], wla_ref[0:64, :],
                  preferred_element_type=jnp.float32)
    acc += jnp.dot(sa1_ref[...] * inva_ref[...], wla_ref[64:128, :],
                   preferred_element_type=jnp.float32)
    acc += jnp.dot(sb0_ref[...] * invb_ref[...], wlb_ref[0:64, :],
                   preferred_element_type=jnp.float32)
    acc += jnp.dot(sb1_ref[...] * invb_ref[...], wlb_ref[64:128, :],
                   preferred_element_type=jnp.float32)
    acc += jnp.dot(x_ref[...], wr_ref[...], preferred_element_type=jnp.float32)
    acc += b_ref[...]
    out_ref[...] = jnp.maximum(acc, 0.0)


def _fused_final_kernel(sa0_ref, sa1_ref, sb0_ref, sb1_ref,
                        inva_ref, invb_ref, x_ref,
                        wla_ref, wlb_ref, wr_ref, b_ref,
                        wlin_ref, blin_ref, out_ref):
    acc = jnp.dot(sa0_ref[...] * inva_ref[...], wla_ref[0:64, :],
                  preferred_element_type=jnp.float32)
    acc += jnp.dot(sa1_ref[...] * inva_ref[...], wla_ref[64:128, :],
                   preferred_element_type=jnp.float32)
    acc += jnp.dot(sb0_ref[...] * invb_ref[...], wlb_ref[0:64, :],
                   preferred_element_type=jnp.float32)
    acc += jnp.dot(sb1_ref[...] * invb_ref[...], wlb_ref[64:128, :],
                   preferred_element_type=jnp.float32)
    acc += jnp.dot(x_ref[...], wr_ref[...], preferred_element_type=jnp.float32)
    acc += b_ref[...]
    h = jnp.maximum(acc, 0.0)
    out_ref[...] = jnp.dot(h, wlin_ref[...],
                           preferred_element_type=jnp.float32) + blin_ref[...]


def _row_spec():
    return pl.BlockSpec((ROW_BLOCK, 128), lambda i: (i, 0))


def _half_spec():
    return pl.BlockSpec((ROW_BLOCK, 64), lambda i: (i, 0))


def _inv_spec():
    return pl.BlockSpec((ROW_BLOCK, 1), lambda i: (i, 0))


def _w_spec():
    return pl.BlockSpec((128, 128), lambda i: (0, 0))


def _b_spec():
    return pl.BlockSpec((1, 128), lambda i: (0, 0))


def _fused_layer(sa0, sa1, sb0, sb1, inva, invb, x, wla, wlb, wr, b):
    return pl.pallas_call(
        _fused_layer_kernel,
        grid=(N_PAD // ROW_BLOCK,),
        in_specs=[_half_spec(), _half_spec(), _half_spec(), _half_spec(),
                  _inv_spec(), _inv_spec(),
                  _row_spec(), _w_spec(), _w_spec(), _w_spec(), _b_spec()],
        out_specs=_row_spec(),
        out_shape=jax.ShapeDtypeStruct((N_PAD, 128), jnp.float32),
    )(sa0, sa1, sb0, sb1, inva, invb, x, wla, wlb, wr, b)


def _fused_final(sa0, sa1, sb0, sb1, inva, invb, x, wla, wlb, wr, b,
                 wlin, blin):
    return pl.pallas_call(
        _fused_final_kernel,
        grid=(N_PAD // ROW_BLOCK,),
        in_specs=[_half_spec(), _half_spec(), _half_spec(), _half_spec(),
                  _inv_spec(), _inv_spec(),
                  _row_spec(), _w_spec(), _w_spec(), _w_spec(), _b_spec(),
                  _w_spec(), _b_spec()],
        out_specs=_row_spec(),
        out_shape=jax.ShapeDtypeStruct((N_PAD, 128), jnp.float32),
    )(sa0, sa1, sb0, sb1, inva, invb, x, wla, wlb, wr, b, wlin, blin)


def kernel(x_note, edge_index_to, edge_index_rev_to, Wl0_to, bl0_to, Wr0_to,
           Wl0_rev, bl0_rev, Wr0_rev, Wl1_to, bl1_to, Wr1_to, Wl1_rev,
           bl1_rev, Wr1_rev, W_lin, b_lin):
    pad_e = E_PAD - E
    src_to = jnp.concatenate([edge_index_to[0], jnp.zeros((pad_e,), jnp.int32)])
    dst_to = jnp.concatenate([edge_index_to[1],
                              jnp.full((pad_e,), 1 << 30, jnp.int32)])
    src_rv = jnp.concatenate([edge_index_rev_to[0],
                              jnp.zeros((pad_e,), jnp.int32)])
    dst_rv = jnp.concatenate([edge_index_rev_to[1],
                              jnp.full((pad_e,), 1 << 30, jnp.int32)])

    agg0 = _make_sc_agg(True)
    agg1 = _make_sc_agg(False)

    x2 = x_note.reshape(2 * N, 64)
    s0_to_h0, s0_to_h1, cnt_to = agg0(x2, src_to, dst_to)
    s0_rv_h0, s0_rv_h1, cnt_rv = agg0(x2, src_rv, dst_rv)

    inv_to = (1.0 / jnp.clip(cnt_to, 1.0, None))[:, None]
    inv_rv = (1.0 / jnp.clip(cnt_rv, 1.0, None))[:, None]

    x_p = jnp.pad(x_note, ((0, N_PAD - N), (0, 0)))
    b0 = (bl0_to + bl0_rev)[None, :]
    h = _fused_layer(s0_to_h0, s0_to_h1, s0_rv_h0, s0_rv_h1,
                     inv_to, inv_rv, x_p,
                     Wl0_to, Wl0_rev, Wr0_to + Wr0_rev, b0)

    h2 = h.reshape(2 * N_PAD, 64)
    s1_to_h0, s1_to_h1, _ = agg1(h2, src_to, dst_to)
    s1_rv_h0, s1_rv_h1, _ = agg1(h2, src_rv, dst_rv)

    b1 = (bl1_to + bl1_rev)[None, :]
    out = _fused_final(s1_to_h0, s1_to_h1, s1_rv_h0, s1_rv_h1,
                       inv_to, inv_rv, h,
                       Wl1_to, Wl1_rev, Wr1_to + Wr1_rev, b1,
                       W_lin, b_lin[None, :])
    return out[:N]
